# baseline (speedup 1.0000x reference)
import functools

import jax
import jax.numpy as jnp
from jax import lax
from jax.experimental import pallas as pl
from jax.experimental.pallas import tpu as pltpu
from jax.experimental.pallas import tpu_sc as plsc

_SCORE_THRESH = 0.05
_NMS_THRESH = 0.5
_DET = 100
_N = 5000
_NTILES = 16
_PER_TILE = 320
_CHUNKS = _PER_TILE // 16
_NEG = float("-inf")
_BIGI = 2**30


def _nms_body(bh, sh, out_h,
              fbox, sbox, sx1, sy1, sx2, sy2,
              msv, areav, rowb, candl, bvv, biv, outv, shared, sem):
    cid = lax.axis_index("c")
    sid = lax.axis_index("s")
    base = jnp.minimum(sid * _PER_TILE, _N - _PER_TILE)
    writer = (cid == 0) & (sid == 0)
    iota = lax.iota(jnp.int32, 16)
    biota = base + iota

    cp = pltpu.async_copy(bh, fbox, sem)
    pltpu.sync_copy(bh.at[pl.ds(base * 4, _PER_TILE * 4)], sbox)
    pltpu.sync_copy(sh.at[pl.ds(base, _PER_TILE)], msv)

    bv = msv[pl.ds(0, 16)]
    bi = biota
    for c in range(_CHUNKS):
        sl = pl.ds(c * 16, 16)
        rows = (c * 16) + iota
        rows4 = rows * 4
        x1 = plsc.load_gather(sbox, [rows4])
        y1 = plsc.load_gather(sbox, [rows4 + 1])
        x2 = plsc.load_gather(sbox, [rows4 + 2])
        y2 = plsc.load_gather(sbox, [rows4 + 3])
        sx1[sl] = x1
        sy1[sl] = y1
        sx2[sl] = x2
        sy2[sl] = y2
        areav[sl] = jnp.maximum(x2 - x1, 0.0) * jnp.maximum(y2 - y1, 0.0)
        if c > 0:
            v = msv[sl]
            take = v > bv
            bv = jnp.where(take, v, bv)
            bi = jnp.where(take, biota + (c * 16), bi)
    bvv[...] = bv
    biv[...] = bi

    @pl.when(writer)
    def _():
        z = jnp.zeros((16,), jnp.float32)
        for r in range(_DET):
            outv[pl.ds(r * 16, 16)] = z

    cp.wait()

    def body(r, carry):
        k, done = carry
        bv = bvv[...]
        bi = biv[...]
        m = jnp.max(bv)
        il = jnp.min(jnp.where(bv == m, bi, _BIGI))

        slot = pl.multiple_of((r % 2) * (_NTILES * 16), _NTILES * 16)
        ilf = plsc.bitcast(jnp.where(iota == 1, il, 0), jnp.float32)
        rowb[...] = jnp.where(iota == 0, m, ilf)
        pltpu.sync_copy(
            rowb, shared.at[pl.ds(slot + pl.multiple_of(sid * 16, 16), 16)])
        plsc.subcore_barrier()
        pltpu.sync_copy(shared.at[pl.ds(slot, _NTILES * 16)], candl)
        vals = plsc.load_gather(candl, [iota * 16])
        idxs = plsc.bitcast(plsc.load_gather(candl, [iota * 16 + 1]), jnp.int32)
        gm = jnp.max(vals)
        gif = jnp.min(jnp.where(vals == gm, idxs, _BIGI))
        done2 = done | (gm <= _SCORE_THRESH)

        @pl.when(jnp.logical_not(done2))
        def _():
            gvec = jnp.full((16,), gif * 4, jnp.int32)
            xi = plsc.load_gather(fbox, [gvec])
            yi = plsc.load_gather(fbox, [gvec + 1])
            Xi = plsc.load_gather(fbox, [gvec + 2])
            Yi = plsc.load_gather(fbox, [gvec + 3])
            ai = jnp.maximum(Xi - xi, 0.0) * jnp.maximum(Yi - yi, 0.0)

            @pl.when(writer)
            def _():
                srow = jnp.full((16,), gm, jnp.float32)
                orow = jnp.where(iota == 0, xi,
                       jnp.where(iota == 1, yi,
                       jnp.where(iota == 2, Xi,
                       jnp.where(iota == 3, Yi,
                       jnp.where(iota == 4, srow, 0.0)))))
                outv[pl.ds(pl.multiple_of(k * 16, 16), 16)] = orow

            nbv = jnp.full((16,), _NEG, jnp.float32)
            nbi = biota
            for c in range(_CHUNKS):
                sl = pl.ds(c * 16, 16)
                xx1 = jnp.maximum(xi, sx1[sl])
                yy1 = jnp.maximum(yi, sy1[sl])
                xx2 = jnp.minimum(Xi, sx2[sl])
                yy2 = jnp.minimum(Yi, sy2[sl])
                inter = jnp.maximum(xx2 - xx1, 0.0) * jnp.maximum(yy2 - yy1, 0.0)
                denom = ((ai + areav[sl]) - inter) + jnp.float32(1e-9)
                iou = inter / denom
                gci = biota + (c * 16)
                sup = (iou > _NMS_THRESH) | (gci == gif)
                msn = jnp.where(sup, _NEG, msv[sl])
                msv[sl] = msn
                take = msn > nbv
                nbv = jnp.where(take, msn, nbv)
                nbi = jnp.where(take, gci, nbi)
            bvv[...] = nbv
            biv[...] = nbi

        knext = k + jnp.where(done2, 0, 1).astype(jnp.int32)
        return (knext, done2)

    lax.fori_loop(0, _DET, body, (jnp.int32(0), jnp.bool_(False)))

    @pl.when(writer)
    def _():
        pltpu.sync_copy(outv, out_h)


_nms_call = functools.partial(
    pl.kernel,
    mesh=plsc.VectorSubcoreMesh(core_axis_name="c", subcore_axis_name="s"),
    out_type=jax.ShapeDtypeStruct((_DET * 16,), jnp.float32),
    compiler_params=pltpu.CompilerParams(needs_layout_passes=False),
    scratch_types=[
        pltpu.VMEM((_N * 4,), jnp.float32),
        pltpu.VMEM((_PER_TILE * 4,), jnp.float32),
        pltpu.VMEM((_PER_TILE,), jnp.float32),
        pltpu.VMEM((_PER_TILE,), jnp.float32),
        pltpu.VMEM((_PER_TILE,), jnp.float32),
        pltpu.VMEM((_PER_TILE,), jnp.float32),
        pltpu.VMEM((_PER_TILE,), jnp.float32),
        pltpu.VMEM((_PER_TILE,), jnp.float32),
        pltpu.VMEM((16,), jnp.float32),
        pltpu.VMEM((_NTILES * 16,), jnp.float32),
        pltpu.VMEM((16,), jnp.float32),
        pltpu.VMEM((16,), jnp.int32),
        pltpu.VMEM((_DET * 16,), jnp.float32),
        pltpu.VMEM_SHARED((2 * _NTILES * 16,), jnp.float32),
        pltpu.SemaphoreType.DMA,
    ],
)


@jax.jit
def kernel(boxes, scores):
    out = _nms_call(_nms_body)(boxes.reshape(-1), scores)
    return out.reshape(_DET, 16)[:, :5]
